# baseline (speedup 1.0000x reference)
import functools

import jax
import jax.numpy as jnp
import numpy as np
from jax.experimental import pallas as pl

_EPS = 0.001
_EPS2 = _EPS * _EPS
_DIL = 3.0
_TMAX = 3.0
_TMIN = float(np.log(0.075))
_LOGEPS = float(np.log(_EPS))


_HLN2 = 0.34657359027997264


def _fast_atan2(y, x):
    ax = jnp.abs(x)
    ay = jnp.abs(y)
    mx = jnp.maximum(ax, ay)
    mn = jnp.minimum(ax, ay)
    a = jnp.exp2(jnp.log2(mn) - jnp.log2(jnp.maximum(mx, 1e-30)))
    s = a * a
    r = a * (0.9998660 + s * (-0.3302995 + s * (0.1801410 + s * (
        -0.0851330 + s * 0.0208351))))
    r = jnp.where(ay > ax, 1.57079637 - r, r)
    r = jnp.where(x < 0, 3.14159274 - r, r)
    return jnp.where(y < 0, -r, r)


def _op_kernel(x_ref, df_ref, d2f_ref, h_ref, la_ref, phi_ref, lt_ref, psi_ref):
    cb = x_ref.shape[1]

    df0c = df_ref[0, :, 0:1]
    df1c = df_ref[0, :, 1:2]
    d2frc = d2f_ref[0, :, 0:1]
    d2fic = d2f_ref[0, :, 1:2]
    edfc = jnp.exp(df0c)
    dfrc = edfc * jnp.cos(df1c)
    dfic = edfc * jnp.sin(df1c)
    inv_mag2 = 1.0 / (edfc * edfc)
    dfac = df0c + float(np.log(2.0))
    dfr2c = 2.0 * dfrc
    dfi2c = 2.0 * dfic
    t2rc = 0.5 * (d2frc * dfrc + d2fic * dfic) * inv_mag2
    t2ic = 0.5 * (d2fic * dfrc - d2frc * dfic) * inv_mag2

    for c in range(cb):
        _channel(x_ref, h_ref, la_ref, phi_ref, lt_ref, psi_ref, c,
                 dfac[c, 0], dfrc[c, 0], dfic[c, 0], dfr2c[c, 0],
                 dfi2c[c, 0], t2rc[c, 0], t2ic[c, 0])


def _channel(x_ref, h_ref, la_ref, phi_ref, lt_ref, psi_ref, c,
             dfa, dfr, dfi, dfr2, dfi2, t2rh, t2ih):
    x = x_ref[0, c]

    xl = jnp.roll(x, -1, axis=1)
    xr = jnp.roll(x, 1, axis=1)
    xu = jnp.roll(x, -1, axis=0)
    xd = jnp.roll(x, 1, axis=0)

    u = xl - xr
    v = xu - xd
    x2 = x + x
    dxx = (xl + xr) - x2
    dyy = (xu + xd) - x2

    u2 = u * u
    v2 = v * v
    uv = u * v
    s2 = u2 + v2
    h = 0.25 * s2
    mask = s2 >= 4.0 * _EPS2

    l2s = jnp.log2(s2)
    ln_alpha = jnp.clip(dfa - _HLN2 * l2s, -_DIL, _DIL)
    la_ref[0, c] = jnp.where(mask, ln_alpha, 0.0)

    phi = _fast_atan2(dfi * u - dfr * v, dfr * u + dfi * v)
    phi_ref[0, c] = jnp.where(mask, phi, 0.0)

    nr = dxx * dfr2 - dyy * dfi2
    ni = dxx * dfi2 + dyy * dfr2
    wr = u2 - v2
    wi = uv + uv
    inv_s22 = jnp.exp2(-2.0 * l2s)
    anr = (nr * wr + ni * wi) * inv_s22 - t2rh
    ani = (ni * wr - nr * wi) * inv_s22 - t2ih

    han = anr * anr + ani * ani
    mask_t = jnp.logical_and(mask, han >= _EPS2)
    ans_r = jnp.where(mask_t, anr, 1.0)
    ans_i = jnp.where(mask_t, ani, 0.0)

    ln_tau = jnp.clip(_HLN2 * jnp.log2(han), _TMIN, _TMAX)
    lt_ref[0, c] = jnp.where(mask_t, ln_tau, _LOGEPS)
    psi = _fast_atan2(ans_i, ans_r)
    psi_ref[0, c] = jnp.where(mask_t, psi, 0.0)

    h_ref[0, c] = h


@functools.partial(jax.jit, static_argnames=("interpret",))
def kernel(x, df, d2f, interpret=False):
    b, C, H, W = x.shape
    CB = 32
    grid = (b, C // CB)
    img_spec = pl.BlockSpec((1, CB, H, W), lambda i, j: (i, j, 0, 0))
    par_spec = pl.BlockSpec((1, CB, 2), lambda i, j: (0, j, 0))
    out = pl.pallas_call(
        _op_kernel,
        grid=grid,
        in_specs=[img_spec, par_spec, par_spec],
        out_specs=[img_spec] * 5,
        out_shape=[jax.ShapeDtypeStruct((b, C, H, W), jnp.float32)] * 5,
        interpret=interpret,
    )(x, df, d2f)
    return tuple(out)


# speedup vs baseline: 1.0978x; 1.0978x over previous
import functools

import jax
import jax.numpy as jnp
import numpy as np
from jax.experimental import pallas as pl

_EPS = 0.001
_EPS2 = _EPS * _EPS
_DIL = 3.0
_TMAX = 3.0
_TMIN = float(np.log(0.075))
_LOGEPS = float(np.log(_EPS))


_HLN2 = 0.34657359027997264


def _fast_atan2(y, x):
    ax = jnp.abs(x)
    ay = jnp.abs(y)
    mx = jnp.maximum(ax, ay)
    mn = jnp.minimum(ax, ay)
    a = mn / jnp.maximum(mx, 1e-30)
    s = a * a
    r = a * (0.9998660 + s * (-0.3302995 + s * (0.1801410 + s * (
        -0.0851330 + s * 0.0208351))))
    r = jnp.where(ay > ax, 1.57079637 - r, r)
    r = jnp.where(x < 0, 3.14159274 - r, r)
    return jnp.where(y < 0, -r, r)


def _op_kernel(x_ref, df_ref, d2f_ref, h_ref, la_ref, phi_ref, lt_ref, psi_ref):
    cb = x_ref.shape[1]

    df0c = df_ref[0, :, 0:1]
    df1c = df_ref[0, :, 1:2]
    d2frc = d2f_ref[0, :, 0:1]
    d2fic = d2f_ref[0, :, 1:2]
    edfc = jnp.exp(df0c)
    dfrc = edfc * jnp.cos(df1c)
    dfic = edfc * jnp.sin(df1c)
    inv_mag2 = 1.0 / (edfc * edfc)
    dfac = df0c + float(np.log(2.0))
    dfr2c = 2.0 * dfrc
    dfi2c = 2.0 * dfic
    t2rc = 0.5 * (d2frc * dfrc + d2fic * dfic) * inv_mag2
    t2ic = 0.5 * (d2fic * dfrc - d2frc * dfic) * inv_mag2

    for c in range(cb):
        _channel(x_ref, h_ref, la_ref, phi_ref, lt_ref, psi_ref, c,
                 dfac[c, 0], dfrc[c, 0], dfic[c, 0], dfr2c[c, 0],
                 dfi2c[c, 0], t2rc[c, 0], t2ic[c, 0])


def _channel(x_ref, h_ref, la_ref, phi_ref, lt_ref, psi_ref, c,
             dfa, dfr, dfi, dfr2, dfi2, t2rh, t2ih):
    x = x_ref[0, c]

    xl = jnp.roll(x, -1, axis=1)
    xr = jnp.roll(x, 1, axis=1)
    xu = jnp.roll(x, -1, axis=0)
    xd = jnp.roll(x, 1, axis=0)

    u = xl - xr
    v = xu - xd
    x2 = x + x
    dxx = (xl + xr) - x2
    dyy = (xu + xd) - x2

    u2 = u * u
    v2 = v * v
    uv = u * v
    s2 = u2 + v2
    h = 0.25 * s2
    mask = s2 >= 4.0 * _EPS2

    l2s = jnp.log2(s2)
    ln_alpha = jnp.clip(dfa - _HLN2 * l2s, -_DIL, _DIL)
    la_ref[0, c] = jnp.where(mask, ln_alpha, 0.0)

    phi = _fast_atan2(dfi * u - dfr * v, dfr * u + dfi * v)
    phi_ref[0, c] = jnp.where(mask, phi, 0.0)

    nr = dxx * dfr2 - dyy * dfi2
    ni = dxx * dfi2 + dyy * dfr2
    wr = u2 - v2
    wi = uv + uv
    inv_s22 = 1.0 / (s2 * s2)
    anr = (nr * wr + ni * wi) * inv_s22 - t2rh
    ani = (ni * wr - nr * wi) * inv_s22 - t2ih

    han = anr * anr + ani * ani
    mask_t = jnp.logical_and(mask, han >= _EPS2)
    ans_r = jnp.where(mask_t, anr, 1.0)
    ans_i = jnp.where(mask_t, ani, 0.0)

    ln_tau = jnp.clip(_HLN2 * jnp.log2(han), _TMIN, _TMAX)
    lt_ref[0, c] = jnp.where(mask_t, ln_tau, _LOGEPS)
    psi = _fast_atan2(ans_i, ans_r)
    psi_ref[0, c] = jnp.where(mask_t, psi, 0.0)

    h_ref[0, c] = h


@functools.partial(jax.jit, static_argnames=("interpret",))
def kernel(x, df, d2f, interpret=False):
    b, C, H, W = x.shape
    CB = 32
    grid = (b, C // CB)
    img_spec = pl.BlockSpec((1, CB, H, W), lambda i, j: (i, j, 0, 0))
    par_spec = pl.BlockSpec((1, CB, 2), lambda i, j: (0, j, 0))
    out = pl.pallas_call(
        _op_kernel,
        grid=grid,
        in_specs=[img_spec, par_spec, par_spec],
        out_specs=[img_spec] * 5,
        out_shape=[jax.ShapeDtypeStruct((b, C, H, W), jnp.float32)] * 5,
        interpret=interpret,
    )(x, df, d2f)
    return tuple(out)


# speedup vs baseline: 1.0983x; 1.0004x over previous
import functools

import jax
import jax.numpy as jnp
import numpy as np
from jax.experimental import pallas as pl
from jax.experimental.pallas import tpu as pltpu

_EPS = 0.001
_EPS2 = _EPS * _EPS
_DIL = 3.0
_TMAX = 3.0
_TMIN = float(np.log(0.075))
_LOGEPS = float(np.log(_EPS))


_HLN2 = 0.34657359027997264


def _fast_atan2(y, x):
    ax = jnp.abs(x)
    ay = jnp.abs(y)
    mx = jnp.maximum(ax, ay)
    mn = jnp.minimum(ax, ay)
    a = mn / jnp.maximum(mx, 1e-30)
    s = a * a
    r = a * (0.9998660 + s * (-0.3302995 + s * (0.1801410 + s * (
        -0.0851330 + s * 0.0208351))))
    r = jnp.where(ay > ax, 1.57079637 - r, r)
    r = jnp.where(x < 0, 3.14159274 - r, r)
    return jnp.where(y < 0, -r, r)


def _op_kernel(x_ref, df_ref, d2f_ref, h_ref, la_ref, phi_ref, lt_ref, psi_ref):
    cb = x_ref.shape[1]

    df0c = df_ref[0, :, 0:1]
    df1c = df_ref[0, :, 1:2]
    d2frc = d2f_ref[0, :, 0:1]
    d2fic = d2f_ref[0, :, 1:2]
    edfc = jnp.exp(df0c)
    dfrc = edfc * jnp.cos(df1c)
    dfic = edfc * jnp.sin(df1c)
    inv_mag2 = 1.0 / (edfc * edfc)
    dfac = df0c + float(np.log(2.0))
    dfr2c = 2.0 * dfrc
    dfi2c = 2.0 * dfic
    t2rc = 0.5 * (d2frc * dfrc + d2fic * dfic) * inv_mag2
    t2ic = 0.5 * (d2fic * dfrc - d2frc * dfic) * inv_mag2

    for c in range(cb):
        _channel(x_ref, h_ref, la_ref, phi_ref, lt_ref, psi_ref, c,
                 dfac[c, 0], dfrc[c, 0], dfic[c, 0], dfr2c[c, 0],
                 dfi2c[c, 0], t2rc[c, 0], t2ic[c, 0])


def _channel(x_ref, h_ref, la_ref, phi_ref, lt_ref, psi_ref, c,
             dfa, dfr, dfi, dfr2, dfi2, t2rh, t2ih):
    x = x_ref[0, c]

    xl = jnp.roll(x, -1, axis=1)
    xr = jnp.roll(x, 1, axis=1)
    xu = jnp.roll(x, -1, axis=0)
    xd = jnp.roll(x, 1, axis=0)

    u = xl - xr
    v = xu - xd
    x2 = x + x
    dxx = (xl + xr) - x2
    dyy = (xu + xd) - x2

    u2 = u * u
    v2 = v * v
    uv = u * v
    s2 = u2 + v2
    h = 0.25 * s2
    mask = s2 >= 4.0 * _EPS2

    l2s = jnp.log2(s2)
    ln_alpha = jnp.clip(dfa - _HLN2 * l2s, -_DIL, _DIL)
    la_ref[0, c] = jnp.where(mask, ln_alpha, 0.0)

    phi = _fast_atan2(dfi * u - dfr * v, dfr * u + dfi * v)
    phi_ref[0, c] = jnp.where(mask, phi, 0.0)

    nr = dxx * dfr2 - dyy * dfi2
    ni = dxx * dfi2 + dyy * dfr2
    wr = u2 - v2
    wi = uv + uv
    inv_s22 = 1.0 / (s2 * s2)
    anr = (nr * wr + ni * wi) * inv_s22 - t2rh
    ani = (ni * wr - nr * wi) * inv_s22 - t2ih

    han = anr * anr + ani * ani
    mask_t = jnp.logical_and(mask, han >= _EPS2)
    ans_r = jnp.where(mask_t, anr, 1.0)
    ans_i = jnp.where(mask_t, ani, 0.0)

    ln_tau = jnp.clip(_HLN2 * jnp.log2(han), _TMIN, _TMAX)
    lt_ref[0, c] = jnp.where(mask_t, ln_tau, _LOGEPS)
    psi = _fast_atan2(ans_i, ans_r)
    psi_ref[0, c] = jnp.where(mask_t, psi, 0.0)

    h_ref[0, c] = h


@functools.partial(jax.jit, static_argnames=("interpret",))
def kernel(x, df, d2f, interpret=False):
    b, C, H, W = x.shape
    CB = 32
    grid = (b, C // CB)
    img_spec = pl.BlockSpec((1, CB, H, W), lambda i, j: (i, j, 0, 0))
    par_spec = pl.BlockSpec((1, CB, 2), lambda i, j: (0, j, 0))
    out = pl.pallas_call(
        _op_kernel,
        grid=grid,
        in_specs=[img_spec, par_spec, par_spec],
        out_specs=[img_spec] * 5,
        out_shape=[jax.ShapeDtypeStruct((b, C, H, W), jnp.float32)] * 5,
        compiler_params=pltpu.CompilerParams(
            dimension_semantics=("parallel", "parallel")),
        interpret=interpret,
    )(x, df, d2f)
    return tuple(out)


# speedup vs baseline: 1.1088x; 1.0096x over previous
import functools

import jax
import jax.numpy as jnp
import numpy as np
from jax.experimental import pallas as pl
from jax.experimental.pallas import tpu as pltpu

_EPS = 0.001
_EPS2 = _EPS * _EPS
_DIL = 3.0
_TMAX = 3.0
_TMIN = float(np.log(0.075))
_LOGEPS = float(np.log(_EPS))


_HLN2 = 0.34657359027997264


def _fast_atan2(y, x):
    ax = jnp.abs(x)
    ay = jnp.abs(y)
    mx = jnp.maximum(ax, ay)
    mn = jnp.minimum(ax, ay)
    a = mn / jnp.maximum(mx, 1e-30)
    s = a * a
    r = a * (0.9998660 + s * (-0.3302995 + s * (0.1801410 + s * (
        -0.0851330 + s * 0.0208351))))
    r = jnp.where(ay > ax, 1.57079637 - r, r)
    r = jnp.where(x < 0, 3.14159274 - r, r)
    return jnp.where(y < 0, -r, r)


def _op_kernel(x_ref, df_ref, d2f_ref, h_ref, la_ref, phi_ref, lt_ref, psi_ref):
    cb = x_ref.shape[1]

    df0c = df_ref[0, :, 0:1]
    df1c = df_ref[0, :, 1:2]
    d2frc = d2f_ref[0, :, 0:1]
    d2fic = d2f_ref[0, :, 1:2]
    edfc = jnp.exp(df0c)
    dfrc = edfc * jnp.cos(df1c)
    dfic = edfc * jnp.sin(df1c)
    inv_mag2 = 1.0 / (edfc * edfc)
    dfac = df0c + float(np.log(2.0))
    dfr2c = 2.0 * dfrc
    dfi2c = 2.0 * dfic
    t2rc = 0.5 * (d2frc * dfrc + d2fic * dfic) * inv_mag2
    t2ic = 0.5 * (d2fic * dfrc - d2frc * dfic) * inv_mag2

    for c in range(cb):
        _channel(x_ref, h_ref, la_ref, phi_ref, lt_ref, psi_ref, c,
                 dfac[c, 0], dfrc[c, 0], dfic[c, 0], dfr2c[c, 0],
                 dfi2c[c, 0], t2rc[c, 0], t2ic[c, 0])


def _channel(x_ref, h_ref, la_ref, phi_ref, lt_ref, psi_ref, c,
             dfa, dfr, dfi, dfr2, dfi2, t2rh, t2ih):
    x = x_ref[0, c]

    xl = jnp.roll(x, -1, axis=1)
    xr = jnp.roll(x, 1, axis=1)
    xu = jnp.roll(x, -1, axis=0)
    xd = jnp.roll(x, 1, axis=0)

    u = xl - xr
    v = xu - xd
    x2 = x + x
    dxx = (xl + xr) - x2
    dyy = (xu + xd) - x2

    u2 = u * u
    v2 = v * v
    uv = u * v
    s2 = u2 + v2
    h = 0.25 * s2
    mask = s2 >= 4.0 * _EPS2

    l2s = jnp.log2(s2)
    ln_alpha = jnp.clip(dfa - _HLN2 * l2s, -_DIL, _DIL)
    la_ref[0, c] = jnp.where(mask, ln_alpha, 0.0)

    phi = _fast_atan2(dfi * u - dfr * v, dfr * u + dfi * v)
    phi_ref[0, c] = jnp.where(mask, phi, 0.0)

    nr = dxx * dfr2 - dyy * dfi2
    ni = dxx * dfi2 + dyy * dfr2
    wr = u2 - v2
    wi = uv + uv
    inv_s22 = 1.0 / (s2 * s2)
    anr = (nr * wr + ni * wi) * inv_s22 - t2rh
    ani = (ni * wr - nr * wi) * inv_s22 - t2ih

    han = anr * anr + ani * ani
    mask_t = jnp.logical_and(mask, han >= _EPS2)
    ans_r = jnp.where(mask_t, anr, 1.0)
    ans_i = jnp.where(mask_t, ani, 0.0)

    ln_tau = jnp.clip(_HLN2 * jnp.log2(han), _TMIN, _TMAX)
    lt_ref[0, c] = jnp.where(mask_t, ln_tau, _LOGEPS)
    psi = _fast_atan2(ans_i, ans_r)
    psi_ref[0, c] = jnp.where(mask_t, psi, 0.0)

    h_ref[0, c] = h


@functools.partial(jax.jit, static_argnames=("interpret",))
def kernel(x, df, d2f, interpret=False):
    b, C, H, W = x.shape
    CB = 64
    grid = (b, C // CB)
    img_spec = pl.BlockSpec((1, CB, H, W), lambda i, j: (i, j, 0, 0))
    par_spec = pl.BlockSpec((1, CB, 2), lambda i, j: (0, j, 0))
    out = pl.pallas_call(
        _op_kernel,
        grid=grid,
        in_specs=[img_spec, par_spec, par_spec],
        out_specs=[img_spec] * 5,
        out_shape=[jax.ShapeDtypeStruct((b, C, H, W), jnp.float32)] * 5,
        compiler_params=pltpu.CompilerParams(
            dimension_semantics=("parallel", "parallel")),
        interpret=interpret,
    )(x, df, d2f)
    return tuple(out)


# speedup vs baseline: 1.1817x; 1.0657x over previous
import functools

import jax
import jax.numpy as jnp
import numpy as np
from jax.experimental import pallas as pl
from jax.experimental.pallas import tpu as pltpu

_EPS = 0.001
_EPS2 = _EPS * _EPS
_DIL = 3.0
_TMAX = 3.0
_TMIN = float(np.log(0.075))
_LOGEPS = float(np.log(_EPS))


_HLN2 = 0.34657359027997264


def _fast_atan2(y, x):
    ax = jnp.abs(x)
    ay = jnp.abs(y)
    mx = jnp.maximum(ax, ay)
    mn = jnp.minimum(ax, ay)
    a = mn / jnp.maximum(mx, 1e-30)
    s = a * a
    r = a * (0.99535841 + s * (-0.28869277 + s * 0.07934162))
    r = jnp.where(ay > ax, 1.57079637 - r, r)
    r = jnp.where(x < 0, 3.14159274 - r, r)
    return jnp.where(y < 0, -r, r)


def _op_kernel(x_ref, df_ref, d2f_ref, h_ref, la_ref, phi_ref, lt_ref, psi_ref):
    cb = x_ref.shape[1]

    df0c = df_ref[0, :, 0:1]
    df1c = df_ref[0, :, 1:2]
    d2frc = d2f_ref[0, :, 0:1]
    d2fic = d2f_ref[0, :, 1:2]
    edfc = jnp.exp(df0c)
    dfrc = edfc * jnp.cos(df1c)
    dfic = edfc * jnp.sin(df1c)
    inv_mag2 = 1.0 / (edfc * edfc)
    dfac = df0c + float(np.log(2.0))
    dfr2c = 2.0 * dfrc
    dfi2c = 2.0 * dfic
    t2rc = 0.5 * (d2frc * dfrc + d2fic * dfic) * inv_mag2
    t2ic = 0.5 * (d2fic * dfrc - d2frc * dfic) * inv_mag2

    for c in range(cb):
        _channel(x_ref, h_ref, la_ref, phi_ref, lt_ref, psi_ref, c,
                 dfac[c, 0], dfrc[c, 0], dfic[c, 0], dfr2c[c, 0],
                 dfi2c[c, 0], t2rc[c, 0], t2ic[c, 0])


def _channel(x_ref, h_ref, la_ref, phi_ref, lt_ref, psi_ref, c,
             dfa, dfr, dfi, dfr2, dfi2, t2rh, t2ih):
    x = x_ref[0, c]

    xl = jnp.roll(x, -1, axis=1)
    xr = jnp.roll(x, 1, axis=1)
    xu = jnp.roll(x, -1, axis=0)
    xd = jnp.roll(x, 1, axis=0)

    u = xl - xr
    v = xu - xd
    x2 = x + x
    dxx = (xl + xr) - x2
    dyy = (xu + xd) - x2

    u2 = u * u
    v2 = v * v
    uv = u * v
    s2 = u2 + v2
    h = 0.25 * s2
    mask = s2 >= 4.0 * _EPS2

    l2s = jnp.log2(s2)
    ln_alpha = jnp.clip(dfa - _HLN2 * l2s, -_DIL, _DIL)
    la_ref[0, c] = jnp.where(mask, ln_alpha, 0.0)

    phi = _fast_atan2(dfi * u - dfr * v, dfr * u + dfi * v)
    phi_ref[0, c] = jnp.where(mask, phi, 0.0)

    nr = dxx * dfr2 - dyy * dfi2
    ni = dxx * dfi2 + dyy * dfr2
    wr = u2 - v2
    wi = uv + uv
    inv_s22 = 1.0 / (s2 * s2)
    anr = (nr * wr + ni * wi) * inv_s22 - t2rh
    ani = (ni * wr - nr * wi) * inv_s22 - t2ih

    han = anr * anr + ani * ani
    mask_t = jnp.logical_and(mask, han >= _EPS2)
    ans_r = jnp.where(mask_t, anr, 1.0)
    ans_i = jnp.where(mask_t, ani, 0.0)

    ln_tau = jnp.clip(_HLN2 * jnp.log2(han), _TMIN, _TMAX)
    lt_ref[0, c] = jnp.where(mask_t, ln_tau, _LOGEPS)
    psi = _fast_atan2(ans_i, ans_r)
    psi_ref[0, c] = jnp.where(mask_t, psi, 0.0)

    h_ref[0, c] = h


@functools.partial(jax.jit, static_argnames=("interpret",))
def kernel(x, df, d2f, interpret=False):
    b, C, H, W = x.shape
    CB = 64
    grid = (b, C // CB)
    img_spec = pl.BlockSpec((1, CB, H, W), lambda i, j: (i, j, 0, 0))
    par_spec = pl.BlockSpec((1, CB, 2), lambda i, j: (0, j, 0))
    out = pl.pallas_call(
        _op_kernel,
        grid=grid,
        in_specs=[img_spec, par_spec, par_spec],
        out_specs=[img_spec] * 5,
        out_shape=[jax.ShapeDtypeStruct((b, C, H, W), jnp.float32)] * 5,
        compiler_params=pltpu.CompilerParams(
            dimension_semantics=("parallel", "parallel")),
        interpret=interpret,
    )(x, df, d2f)
    return tuple(out)


# speedup vs baseline: 1.2014x; 1.0167x over previous
import functools

import jax
import jax.numpy as jnp
import numpy as np
from jax.experimental import pallas as pl
from jax.experimental.pallas import tpu as pltpu

_EPS = 0.001
_EPS2 = _EPS * _EPS
_DIL = 3.0
_TMAX = 3.0
_TMIN = float(np.log(0.075))
_LOGEPS = float(np.log(_EPS))


_HLN2 = 0.34657359027997264


def _fast_atan2(y, x):
    ax = jnp.abs(x)
    ay = jnp.abs(y)
    mx = jnp.maximum(ax, ay)
    mn = jnp.minimum(ax, ay)
    a = mn / jnp.maximum(mx, 1e-30)
    s = a * a
    r = a * (0.99535841 + s * (-0.28869277 + s * 0.07934162))
    r = jnp.where(ay > ax, 1.57079637 - r, r)
    r = jnp.where(x < 0, 3.14159274 - r, r)
    return jnp.where(y < 0, -r, r)


def _op_kernel(x_ref, df_ref, d2f_ref, h_ref, la_ref, phi_ref, lt_ref, psi_ref):
    cb = x_ref.shape[1]

    df0c = df_ref[0, :, 0:1]
    df1c = df_ref[0, :, 1:2]
    d2frc = d2f_ref[0, :, 0:1]
    d2fic = d2f_ref[0, :, 1:2]
    edfc = jnp.exp(df0c)
    dfrc = edfc * jnp.cos(df1c)
    dfic = edfc * jnp.sin(df1c)
    inv_mag2 = 1.0 / (edfc * edfc)
    dfac = df0c + float(np.log(2.0))
    dfr2c = 2.0 * dfrc
    dfi2c = 2.0 * dfic
    t2rc = 0.5 * (d2frc * dfrc + d2fic * dfic) * inv_mag2
    t2ic = 0.5 * (d2fic * dfrc - d2frc * dfic) * inv_mag2

    for c in range(cb):
        _channel(x_ref, h_ref, la_ref, phi_ref, lt_ref, psi_ref, c,
                 dfac[c, 0], dfrc[c, 0], dfic[c, 0], dfr2c[c, 0],
                 dfi2c[c, 0], t2rc[c, 0], t2ic[c, 0])


def _channel(x_ref, h_ref, la_ref, phi_ref, lt_ref, psi_ref, c,
             dfa, dfr, dfi, dfr2, dfi2, t2rh, t2ih):
    x = x_ref[0, c]

    xl = jnp.roll(x, -1, axis=1)
    xr = jnp.roll(x, 1, axis=1)
    xu = jnp.roll(x, -1, axis=0)
    xd = jnp.roll(x, 1, axis=0)

    u = xl - xr
    v = xu - xd
    x2 = x + x
    dxx = (xl + xr) - x2
    dyy = (xu + xd) - x2

    u2 = u * u
    v2 = v * v
    uv = u * v
    s2 = u2 + v2
    h = 0.25 * s2
    mask = s2 >= 4.0 * _EPS2

    l2s = jnp.log2(s2)
    ln_alpha = jnp.clip(dfa - _HLN2 * l2s, -_DIL, _DIL)
    la_ref[0, c] = jnp.where(mask, ln_alpha, 0.0)

    phi = _fast_atan2(dfi * u - dfr * v, dfr * u + dfi * v)
    phi_ref[0, c] = jnp.where(mask, phi, 0.0)

    nr = dxx * dfr2 - dyy * dfi2
    ni = dxx * dfi2 + dyy * dfr2
    wr = u2 - v2
    wi = uv + uv
    inv_s22 = 1.0 / (s2 * s2)
    anr = (nr * wr + ni * wi) * inv_s22 - t2rh
    ani = (ni * wr - nr * wi) * inv_s22 - t2ih

    han = anr * anr + ani * ani
    mask_t = jnp.logical_and(mask, han >= _EPS2)

    ln_tau = jnp.clip(_HLN2 * jnp.log2(han), _TMIN, _TMAX)
    lt_ref[0, c] = jnp.where(mask_t, ln_tau, _LOGEPS)
    psi = _fast_atan2(ani, anr)
    psi_ref[0, c] = jnp.where(mask_t, psi, 0.0)

    h_ref[0, c] = h


@functools.partial(jax.jit, static_argnames=("interpret",))
def kernel(x, df, d2f, interpret=False):
    b, C, H, W = x.shape
    CB = 64
    grid = (b, C // CB)
    img_spec = pl.BlockSpec((1, CB, H, W), lambda i, j: (i, j, 0, 0))
    par_spec = pl.BlockSpec((1, CB, 2), lambda i, j: (0, j, 0))
    out = pl.pallas_call(
        _op_kernel,
        grid=grid,
        in_specs=[img_spec, par_spec, par_spec],
        out_specs=[img_spec] * 5,
        out_shape=[jax.ShapeDtypeStruct((b, C, H, W), jnp.float32)] * 5,
        compiler_params=pltpu.CompilerParams(
            dimension_semantics=("parallel", "parallel")),
        interpret=interpret,
    )(x, df, d2f)
    return tuple(out)


# speedup vs baseline: 1.2082x; 1.0056x over previous
import functools

import jax
import jax.numpy as jnp
import numpy as np
from jax.experimental import pallas as pl
from jax.experimental.pallas import tpu as pltpu

_EPS = 0.001
_EPS2 = _EPS * _EPS
_DIL = 3.0
_TMAX = 3.0
_TMIN = float(np.log(0.075))
_LOGEPS = float(np.log(_EPS))


_HLN2 = 0.34657359027997264


def _fast_atan2(y, x, guard=True):
    ax = jnp.abs(x)
    ay = jnp.abs(y)
    mx = jnp.maximum(ax, ay)
    mn = jnp.minimum(ax, ay)
    a = mn / (jnp.maximum(mx, 1e-30) if guard else mx)
    s = a * a
    r = a * (0.99535841 + s * (-0.28869277 + s * 0.07934162))
    r = jnp.where(ay > ax, 1.57079637 - r, r)
    r = jnp.where(x < 0, 3.14159274 - r, r)
    return jnp.where(y < 0, -r, r)


def _op_kernel(x_ref, df_ref, d2f_ref, h_ref, la_ref, phi_ref, lt_ref, psi_ref):
    cb = x_ref.shape[1]

    df0c = df_ref[0, :, 0:1]
    df1c = df_ref[0, :, 1:2]
    d2frc = d2f_ref[0, :, 0:1]
    d2fic = d2f_ref[0, :, 1:2]
    edfc = jnp.exp(df0c)
    dfrc = edfc * jnp.cos(df1c)
    dfic = edfc * jnp.sin(df1c)
    inv_mag2 = 1.0 / (edfc * edfc)
    dfac = df0c + float(np.log(2.0))
    dfr2c = 2.0 * dfrc
    dfi2c = 2.0 * dfic
    t2rc = 0.5 * (d2frc * dfrc + d2fic * dfic) * inv_mag2
    t2ic = 0.5 * (d2fic * dfrc - d2frc * dfic) * inv_mag2

    for c in range(cb):
        _channel(x_ref, h_ref, la_ref, phi_ref, lt_ref, psi_ref, c,
                 dfac[c, 0], dfrc[c, 0], dfic[c, 0], dfr2c[c, 0],
                 dfi2c[c, 0], t2rc[c, 0], t2ic[c, 0])


def _channel(x_ref, h_ref, la_ref, phi_ref, lt_ref, psi_ref, c,
             dfa, dfr, dfi, dfr2, dfi2, t2rh, t2ih):
    x = x_ref[0, c]

    xl = jnp.roll(x, -1, axis=1)
    xr = jnp.roll(x, 1, axis=1)
    xu = jnp.roll(x, -1, axis=0)
    xd = jnp.roll(x, 1, axis=0)

    u = xl - xr
    v = xu - xd
    x2 = x + x
    dxx = (xl + xr) - x2
    dyy = (xu + xd) - x2

    u2 = u * u
    v2 = v * v
    uv = u * v
    s2 = u2 + v2
    h = 0.25 * s2
    mask = s2 >= 4.0 * _EPS2

    l2s = jnp.log2(s2)
    ln_alpha = jnp.clip(dfa - _HLN2 * l2s, -_DIL, _DIL)
    la_ref[0, c] = jnp.where(mask, ln_alpha, 0.0)

    phi = _fast_atan2(dfi * u - dfr * v, dfr * u + dfi * v)
    phi_ref[0, c] = jnp.where(mask, phi, 0.0)

    nr = dxx * dfr2 - dyy * dfi2
    ni = dxx * dfi2 + dyy * dfr2
    wr = u2 - v2
    wi = uv + uv
    inv_s22 = 1.0 / (s2 * s2)
    anr = (nr * wr + ni * wi) * inv_s22 - t2rh
    ani = (ni * wr - nr * wi) * inv_s22 - t2ih

    han = anr * anr + ani * ani
    mask_t = jnp.logical_and(mask, han >= _EPS2)

    ln_tau = jnp.clip(_HLN2 * jnp.log2(han), _TMIN, _TMAX)
    lt_ref[0, c] = jnp.where(mask_t, ln_tau, _LOGEPS)
    psi = _fast_atan2(ani, anr, guard=False)
    psi_ref[0, c] = jnp.where(mask_t, psi, 0.0)

    h_ref[0, c] = h


@functools.partial(jax.jit, static_argnames=("interpret",))
def kernel(x, df, d2f, interpret=False):
    b, C, H, W = x.shape
    CB = 64
    grid = (b, C // CB)
    img_spec = pl.BlockSpec((1, CB, H, W), lambda i, j: (i, j, 0, 0))
    par_spec = pl.BlockSpec((1, CB, 2), lambda i, j: (0, j, 0))
    out = pl.pallas_call(
        _op_kernel,
        grid=grid,
        in_specs=[img_spec, par_spec, par_spec],
        out_specs=[img_spec] * 5,
        out_shape=[jax.ShapeDtypeStruct((b, C, H, W), jnp.float32)] * 5,
        compiler_params=pltpu.CompilerParams(
            dimension_semantics=("parallel", "parallel")),
        interpret=interpret,
    )(x, df, d2f)
    return tuple(out)
